# baseline (speedup 1.0000x reference)
import functools

import jax
import jax.numpy as jnp
from jax import lax
from jax.experimental import pallas as pl
from jax.experimental.pallas import tpu as pltpu
from jax.experimental.pallas import tpu_sc as plsc

B = 16384
VOCAB = 10
DIM = 50
EPS = 1e-5

NC = 2
NS = 16
NW = NC * NS
BPW = B // NW
HPW = 2 * BPW
NBLK = BPW // 16
MAGIC = 0x5F3759DF


@functools.cache
def _make_fused_call():
    @functools.partial(
        pl.kernel,
        mesh=plsc.VectorSubcoreMesh(core_axis_name="c", subcore_axis_name="s"),
        out_type=jax.ShapeDtypeStruct((DIM, B), jnp.float32),
        scratch_types=[
            pltpu.VMEM((VOCAB, DIM), jnp.float32),
            pltpu.VMEM((128,), jnp.float32),
            pltpu.VMEM((HPW,), jnp.int32),
            pltpu.VMEM((16,), jnp.float32),
            pltpu.VMEM((NS, 16), jnp.float32),
            pltpu.VMEM_SHARED((NS, 16), jnp.float32),
            pltpu.VMEM((DIM, BPW), jnp.float32),
        ],
        compiler_params=pltpu.CompilerParams(needs_layout_passes=False),
    )
    def _fused(tbl_hbm, idx_hbm, gb_hbm, out_hbm,
               tbl_v, gb_v, idx_v, hist_v, hall_v, shared, out_v):
        c = lax.axis_index("c")
        s = lax.axis_index("s")
        wid = s * NC + c
        lanes = lax.iota(jnp.int32, 16)

        pltpu.sync_copy(tbl_hbm, tbl_v)
        pltpu.sync_copy(gb_hbm, gb_v)
        pltpu.sync_copy(idx_hbm.at[pl.ds(s * HPW, HPW)], idx_v)

        def hist_body(k, accs):
            xv = idx_v[pl.ds(k * 16, 16)]
            return tuple(
                acc + jnp.where(xv == v, 1.0, 0.0).astype(jnp.float32)
                for v, acc in enumerate(accs)
            )

        accs = lax.fori_loop(
            0, HPW // 16, hist_body,
            tuple(jnp.zeros((16,), jnp.float32) for _ in range(VOCAB)),
        )
        hvec = jnp.zeros((16,), jnp.float32)
        for v in range(VOCAB):
            cv = jnp.sum(accs[v])
            hvec = jnp.where(lanes == v, jnp.full((16,), cv), hvec)
        hist_v[...] = hvec

        pltpu.sync_copy(hist_v, shared.at[s])
        plsc.subcore_barrier()
        pltpu.sync_copy(shared, hall_v)
        cnt = jnp.zeros((16,), jnp.float32)
        for t in range(NS):
            cnt = cnt + hall_v[t, :]
        w = cnt * (1.0 / B)

        gvecs = [gb_v[pl.ds(k * 16, 16)] for k in range(4)]
        bvecs = [gb_v[pl.ds(64 + k * 16, 16)] for k in range(4)]
        vmask = lanes < VOCAB
        cols = []
        for d in range(DIM):
            dsp = jnp.full((16,), d, jnp.int32)
            col = plsc.load_gather(tbl_v, [lanes, dsp], mask=vmask)
            mean = jnp.full((16,), jnp.sum(w * col))
            e = col - mean
            va = jnp.full((16,), jnp.sum(w * e * e) + EPS)
            y = plsc.bitcast(
                MAGIC - (plsc.bitcast(va, jnp.int32) >> 1), jnp.float32
            )
            for _ in range(3):
                y = y * (1.5 - 0.5 * va * y * y)
            gd = jnp.full((16,), gvecs[d // 16][d % 16])
            bd = jnp.full((16,), bvecs[d // 16][d % 16])
            cols.append(jnp.maximum(e * y * gd + bd, 0.0))

        dnums = lax.GatherDimensionNumbers(
            offset_dims=(), collapsed_slice_dims=(0,), start_index_map=(0,)
        )
        base = c * BPW

        @plsc.parallel_loop(0, NBLK)
        def blk_body(jblk):
            x16 = idx_v[pl.ds(base + jblk * 16, 16)]
            xi = x16[:, None]
            for d in range(DIM):
                out_v[d, pl.ds(jblk * 16, 16)] = lax.gather(
                    cols[d], xi, dnums, (1,),
                    mode=lax.GatherScatterMode.PROMISE_IN_BOUNDS,
                )

        pltpu.sync_copy(out_v, out_hbm.at[:, pl.ds(wid * BPW, BPW)])

    return _fused


def kernel(x, table, gamma, beta):
    x = x.astype(jnp.int32)
    pad = jnp.zeros((14,), jnp.float32)
    gb = jnp.concatenate([gamma, pad, beta, pad])
    return _make_fused_call()(table, x, gb).T


# speedup vs baseline: 1.0810x; 1.0810x over previous
import functools

import jax
import jax.numpy as jnp
from jax import lax
from jax.experimental import pallas as pl
from jax.experimental.pallas import tpu as pltpu
from jax.experimental.pallas import tpu_sc as plsc

B = 16384
VOCAB = 10
DIM = 50
EPS = 1e-5

NC = 2
NS = 16
NW = NC * NS
BPW = B // NW
NBLK = BPW // 16


def _stats_body(x_ref, tbl_ref, gamma_ref, beta_ref, out_ref):
    x = x_ref[...]
    tbl = tbl_ref[...]
    counts = [jnp.sum((x == v).astype(jnp.float32)) for v in range(VOCAB)]
    inv_b = 1.0 / B
    mean = jnp.zeros((1, DIM), jnp.float32)
    for v in range(VOCAB):
        mean = mean + (counts[v] * inv_b) * tbl[v : v + 1, :]
    var = jnp.zeros((1, DIM), jnp.float32)
    for v in range(VOCAB):
        d = tbl[v : v + 1, :] - mean
        var = var + (counts[v] * inv_b) * (d * d)
    scale = gamma_ref[...] * lax.rsqrt(var + EPS)
    normed = jnp.maximum((tbl - mean) * scale + beta_ref[...], 0.0)
    out_ref[:, :VOCAB] = normed.T
    out_ref[:, VOCAB:] = jnp.zeros((DIM, 16 - VOCAB), jnp.float32)


_stats_call = pl.pallas_call(
    _stats_body,
    out_shape=jax.ShapeDtypeStruct((DIM, 16), jnp.float32),
)


@functools.cache
def _make_gather_call():
    @functools.partial(
        pl.kernel,
        mesh=plsc.VectorSubcoreMesh(core_axis_name="c", subcore_axis_name="s"),
        out_type=jax.ShapeDtypeStruct((DIM, B), jnp.float32),
        scratch_types=[
            pltpu.VMEM((DIM, 16), jnp.float32),
            pltpu.VMEM((BPW,), jnp.int32),
            pltpu.VMEM((DIM, BPW), jnp.float32),
            pltpu.SemaphoreType.DMA,
        ],
        compiler_params=pltpu.CompilerParams(needs_layout_passes=False),
    )
    def _gather_call(tbl_hbm, idx_hbm, out_hbm, tbl_v, idx_v, out_v, sem):
        wid = lax.axis_index("s") * NC + lax.axis_index("c")
        pltpu.sync_copy(tbl_hbm, tbl_v)
        pltpu.sync_copy(idx_hbm.at[pl.ds(wid * BPW, BPW)], idx_v)
        cols = [tbl_v[d, :] for d in range(DIM)]
        dnums = lax.GatherDimensionNumbers(
            offset_dims=(), collapsed_slice_dims=(0,), start_index_map=(0,)
        )
        half = BPW // 2
        copies = []
        for jblk in range(NBLK):
            x16 = idx_v[pl.ds(jblk * 16, 16)]
            xi = x16[:, None]
            for d in range(DIM):
                out_v[d, pl.ds(jblk * 16, 16)] = lax.gather(
                    cols[d],
                    xi,
                    dnums,
                    (1,),
                    mode=lax.GatherScatterMode.PROMISE_IN_BOUNDS,
                )
            if jblk == NBLK // 2 - 1:
                copies.append(
                    pltpu.async_copy(
                        out_v.at[:, pl.ds(0, half)],
                        out_hbm.at[:, pl.ds(wid * BPW, half)],
                        sem,
                    )
                )
        copies.append(
            pltpu.async_copy(
                out_v.at[:, pl.ds(half, half)],
                out_hbm.at[:, pl.ds(wid * BPW + half, half)],
                sem,
            )
        )
        for cp in copies:
            cp.wait()

    return _gather_call


def kernel(x, table, gamma, beta):
    x = x.astype(jnp.int32)
    tbl_t = _stats_call(
        x,
        table,
        gamma.reshape(1, DIM),
        beta.reshape(1, DIM),
    )
    return _make_gather_call()(tbl_t, x).T


# speedup vs baseline: 1.0865x; 1.0051x over previous
import functools

import jax
import jax.numpy as jnp
from jax import lax
from jax.experimental import pallas as pl
from jax.experimental.pallas import tpu as pltpu
from jax.experimental.pallas import tpu_sc as plsc

B = 16384
VOCAB = 10
DIM = 50
EPS = 1e-5

NC = 2
NS = 16
NW = NC * NS
BPW = B // NW
NBLK = BPW // 16


def _stats_body(x_hbm, tbl_ref, gamma_ref, beta_ref, out_ref, x_vmem, sem):
    pltpu.make_async_copy(x_hbm, x_vmem, sem).start()
    tbl = tbl_ref[...]
    pltpu.make_async_copy(x_hbm, x_vmem, sem).wait()
    x = x_vmem[...]
    counts = [jnp.sum((x == v).astype(jnp.float32)) for v in range(VOCAB)]
    inv_b = 1.0 / B
    mean = jnp.zeros((1, DIM), jnp.float32)
    for v in range(VOCAB):
        mean = mean + (counts[v] * inv_b) * tbl[v : v + 1, :]
    var = jnp.zeros((1, DIM), jnp.float32)
    for v in range(VOCAB):
        d = tbl[v : v + 1, :] - mean
        var = var + (counts[v] * inv_b) * (d * d)
    scale = gamma_ref[...] * lax.rsqrt(var + EPS)
    normed = jnp.maximum((tbl - mean) * scale + beta_ref[...], 0.0)
    out_ref[:, :VOCAB] = normed.T
    out_ref[:, VOCAB:] = jnp.zeros((DIM, 16 - VOCAB), jnp.float32)


_stats_call = pl.pallas_call(
    _stats_body,
    out_shape=jax.ShapeDtypeStruct((DIM, 16), jnp.float32),
    in_specs=[
        pl.BlockSpec(memory_space=pl.ANY),
        pl.BlockSpec(memory_space=pltpu.VMEM),
        pl.BlockSpec(memory_space=pltpu.VMEM),
        pl.BlockSpec(memory_space=pltpu.VMEM),
    ],
    scratch_shapes=[
        pltpu.VMEM((B,), jnp.int32),
        pltpu.SemaphoreType.DMA,
    ],
)


@functools.cache
def _make_gather_call():
    @functools.partial(
        pl.kernel,
        mesh=plsc.VectorSubcoreMesh(core_axis_name="c", subcore_axis_name="s"),
        out_type=jax.ShapeDtypeStruct((DIM, B), jnp.float32),
        scratch_types=[
            pltpu.VMEM((DIM, 16), jnp.float32),
            pltpu.VMEM((BPW,), jnp.int32),
            pltpu.VMEM((DIM, BPW), jnp.float32),
            pltpu.SemaphoreType.DMA,
        ],
        compiler_params=pltpu.CompilerParams(needs_layout_passes=False),
    )
    def _gather_call(tbl_hbm, idx_hbm, out_hbm, tbl_v, idx_v, out_v, sem):
        wid = lax.axis_index("s") * NC + lax.axis_index("c")
        pltpu.sync_copy(tbl_hbm, tbl_v)
        pltpu.sync_copy(idx_hbm.at[pl.ds(wid * BPW, BPW)], idx_v)
        cols = [tbl_v[d, :] for d in range(DIM)]
        dnums = lax.GatherDimensionNumbers(
            offset_dims=(), collapsed_slice_dims=(0,), start_index_map=(0,)
        )
        half = BPW // 2
        copies = []
        for jblk in range(NBLK):
            x16 = idx_v[pl.ds(jblk * 16, 16)]
            xi = x16[:, None]
            for d in range(DIM):
                out_v[d, pl.ds(jblk * 16, 16)] = lax.gather(
                    cols[d],
                    xi,
                    dnums,
                    (1,),
                    mode=lax.GatherScatterMode.PROMISE_IN_BOUNDS,
                )
            if jblk == NBLK // 2 - 1:
                copies.append(
                    pltpu.async_copy(
                        out_v.at[:, pl.ds(0, half)],
                        out_hbm.at[:, pl.ds(wid * BPW, half)],
                        sem,
                    )
                )
        copies.append(
            pltpu.async_copy(
                out_v.at[:, pl.ds(half, half)],
                out_hbm.at[:, pl.ds(wid * BPW + half, half)],
                sem,
            )
        )
        for cp in copies:
            cp.wait()

    return _gather_call


def kernel(x, table, gamma, beta):
    x = x.astype(jnp.int32)
    tbl_t = _stats_call(
        x,
        table,
        gamma.reshape(1, DIM),
        beta.reshape(1, DIM),
    )
    return _make_gather_call()(tbl_t, x).T
